# baseline (speedup 1.0000x reference)
import functools

import jax
import jax.numpy as jnp
from jax import lax
from jax.experimental import pallas as pl
from jax.experimental.pallas import tpu as pltpu
from jax.experimental.pallas import tpu_sc as plsc

N = 50000
K = 16
H = 3
F = 6
E = N * K
D = H * F

NC = 2
NS = 16
NW = NC * NS
CH = 128
ROWS = -(-E // (NW * CH))
EP = NW * ROWS * CH

EPS = 1e-5


def _r16(v):
    return v.astype(jnp.bfloat16).astype(jnp.float32)


def _gather_body(t32, td, sidx, didx, g1, g2, idx1_v, idx2_v,
                 r1a, r2a, r1b, r2b, sem_a, sem_b):
    c = lax.axis_index("c")
    s = lax.axis_index("s")
    wid = s * NC + c
    pltpu.sync_copy(sidx.at[wid], idx1_v)
    pltpu.sync_copy(didx.at[wid], idx2_v)

    def fire(j, r1, r2, sem):
        pltpu.async_copy(t32.at[idx1_v.at[j]], r1, sem)
        pltpu.async_copy(td.at[idx2_v.at[j]], r2, sem)

    def drain_wb(j, r1, r2, sem):
        pltpu.make_async_copy(t32.at[idx1_v.at[j]], r1, sem).wait()
        pltpu.make_async_copy(td.at[idx2_v.at[j]], r2, sem).wait()
        base = (wid * ROWS + j) * CH
        pltpu.sync_copy(r1, g1.at[pl.ds(base, CH)])
        pltpu.sync_copy(r2, g2.at[pl.ds(base, CH)])

    fire(0, r1a, r2a, sem_a)

    @pl.loop(0, ROWS, step=2)
    def _(j):
        fire(j + 1, r1b, r2b, sem_b)
        drain_wb(j, r1a, r2a, sem_a)

        @pl.when(j + 2 < ROWS)
        def _():
            fire(j + 2, r1a, r2a, sem_a)

        drain_wb(j + 1, r1b, r2b, sem_b)


@functools.cache
def _make_gather():
    return pl.kernel(
        _gather_body,
        out_type=(jax.ShapeDtypeStruct((EP, 32), jnp.float32),
                  jax.ShapeDtypeStruct((EP, 8), jnp.float32)),
        mesh=plsc.VectorSubcoreMesh(core_axis_name="c", subcore_axis_name="s"),
        scratch_types=[
            pltpu.VMEM((ROWS, CH), jnp.int32),
            pltpu.VMEM((ROWS, CH), jnp.int32),
            pltpu.VMEM((CH, 32), jnp.float32),
            pltpu.VMEM((CH, 8), jnp.float32),
            pltpu.VMEM((CH, 32), jnp.float32),
            pltpu.VMEM((CH, 8), jnp.float32),
            pltpu.SemaphoreType.DMA,
            pltpu.SemaphoreType.DMA,
        ],
        compiler_params=pltpu.CompilerParams(use_tc_tiling_on_sc=False),
    )


def _gather_call(t32, td, sidx, didx):
    s = sidx.reshape(-1)
    d = didx.reshape(-1)
    return t32[s], td[d]


_BB = 2000
_BP = 400


def _stats_body(x_ref, s_ref, q_ref):
    @pl.when(pl.program_id(0) == 0)
    def _():
        s_ref[...] = jnp.zeros_like(s_ref)
        q_ref[...] = jnp.zeros_like(q_ref)

    xb = x_ref[...]
    s_ref[...] += jnp.sum(xb, axis=0, keepdims=True)
    q_ref[...] += jnp.sum(xb * xb, axis=0, keepdims=True)


def _stats_call(x):
    return pl.pallas_call(
        _stats_body,
        grid=(N // _BB,),
        in_specs=[pl.BlockSpec((_BB, 1), lambda i: (i, 0))],
        out_specs=[pl.BlockSpec((1, 1), lambda i: (0, 0)),
                   pl.BlockSpec((1, 1), lambda i: (0, 0))],
        out_shape=[jax.ShapeDtypeStruct((1, 1), jnp.float32),
                   jax.ShapeDtypeStruct((1, 1), jnp.float32)],
    )(x)


def _dot_hi(a, b):
    return jnp.dot(a, b, preferred_element_type=jnp.float32,
                   precision=lax.Precision.HIGHEST)


def _dot_bf(a, b):
    return jnp.dot(a, b, preferred_element_type=jnp.float32)


def _pre0_body(x_ref, a_ref, wx_ref, b_ref, Ar_ref, Adr_ref,
               x0_ref, t_ref, td_ref):
    x0 = x_ref[...] * a_ref[0] + a_ref[1]
    x0_ref[...] = x0
    xe = x0 * wx_ref[...] + b_ref[...]
    xeh = _r16(xe)
    t_ref[...] = jnp.concatenate([xe, _dot_bf(xeh, Ar_ref[...])], axis=1)
    td_ref[...] = _dot_bf(xeh, Adr_ref[...])


def _pre0_call(x, a, wx, b, Ar, Adr):
    return pl.pallas_call(
        _pre0_body,
        grid=(N // _BB,),
        in_specs=[
            pl.BlockSpec((_BB, 1), lambda i: (i, 0)),
            pl.BlockSpec(memory_space=pltpu.SMEM),
            pl.BlockSpec((1, D), lambda i: (0, 0)),
            pl.BlockSpec((1, D), lambda i: (0, 0)),
            pl.BlockSpec((D, 14), lambda i: (0, 0)),
            pl.BlockSpec((D, 8), lambda i: (0, 0)),
        ],
        out_specs=[pl.BlockSpec((_BB, 1), lambda i: (i, 0)),
                   pl.BlockSpec((_BB, 32), lambda i: (i, 0)),
                   pl.BlockSpec((_BB, 8), lambda i: (i, 0))],
        out_shape=[jax.ShapeDtypeStruct((N, 1), jnp.float32),
                   jax.ShapeDtypeStruct((N, 32), jnp.float32),
                   jax.ShapeDtypeStruct((N, 8), jnp.float32)],
    )(x, a, wx, b, Ar, Adr)


def _pre_body(nf, refs):
    i = 0
    x0_ref = refs[i]; i += 1
    y_refs = refs[i:i + nf]; i += nf
    ss_refs = refs[i:i + 2 * nf]; i += 2 * nf
    wxr_ref = refs[i]; i += 1
    w_refs = refs[i:i + nf]; i += nf
    b_ref = refs[i]; i += 1
    Ar_ref = refs[i]; Adr_ref = refs[i + 1]; i += 2
    t_ref = refs[i]; td_ref = refs[i + 1]
    xe = _r16(x0_ref[...]) * wxr_ref[...] + b_ref[...]
    for g in range(nf):
        yn = y_refs[g][...] * ss_refs[2 * g][...] + ss_refs[2 * g + 1][...]
        xe += _dot_bf(_r16(yn), w_refs[g][...])
    xeh = _r16(xe)
    t_ref[...] = jnp.concatenate([xe, _dot_bf(xeh, Ar_ref[...])], axis=1)
    td_ref[...] = _dot_bf(xeh, Adr_ref[...])


def _pre_call(nf, x0, ys, sss, wxr, ws, b, Ar, Adr):
    body = lambda *refs: _pre_body(nf, refs)
    vec = pl.BlockSpec((1, D), lambda i: (0, 0))
    return pl.pallas_call(
        body,
        grid=(N // _BB,),
        in_specs=(
            [pl.BlockSpec((_BB, 1), lambda i: (i, 0))]
            + [pl.BlockSpec((_BB, D), lambda i: (i, 0))] * nf
            + [vec] * (2 * nf)
            + [vec]
            + [pl.BlockSpec((D, D), lambda i: (0, 0))] * nf
            + [vec,
               pl.BlockSpec((D, 14), lambda i: (0, 0)),
               pl.BlockSpec((D, 8), lambda i: (0, 0))]
        ),
        out_specs=[pl.BlockSpec((_BB, 32), lambda i: (i, 0)),
                   pl.BlockSpec((_BB, 8), lambda i: (i, 0))],
        out_shape=[jax.ShapeDtypeStruct((N, 32), jnp.float32),
                   jax.ShapeDtypeStruct((N, 8), jnp.float32)],
    )(x0, *ys, *sss, wxr, *ws, b, Ar, Adr)


def _post_body(has_e, refs):
    if has_e:
        g1_ref, g2_ref, e0_ref, e1_ref, R_ref, p_ref, y_ref, s_ref, q_ref = refs
    else:
        g1_ref, g2_ref, R_ref, p_ref, y_ref, s_ref, q_ref = refs
    g1 = g1_ref[...]
    att = g1[:, :, D:D + H] + g2_ref[...][:, :, 0:3] + p_ref[0]
    if has_e:
        ae = _r16(e0_ref[...]) * p_ref[1] + _r16(e1_ref[...]) * p_ref[2]
        att += ae[:, :, None]
    m = jnp.max(att, axis=1, keepdims=True)
    w = jnp.exp(att - m)
    wn = w / jnp.sum(w, axis=1, keepdims=True)
    wfull = jax.lax.dot_general(
        wn, R_ref[...], (((2,), (0,)), ((), ())),
        preferred_element_type=jnp.float32,
        precision=lax.Precision.HIGHEST)
    y = jnp.sum(wfull * g1, axis=1)[:, 0:D] * (1.0 / K)
    y_ref[...] = y

    @pl.when(pl.program_id(0) == 0)
    def _():
        s_ref[...] = jnp.zeros_like(s_ref)
        q_ref[...] = jnp.zeros_like(q_ref)

    s_ref[...] += jnp.sum(y, axis=0, keepdims=True)
    q_ref[...] += jnp.sum(y * y, axis=0, keepdims=True)


def _post_call(g1, g2, e, p):
    has_e = e is not None
    body = lambda *refs: _post_body(has_e, refs)
    in_specs = [pl.BlockSpec((_BP, K, 32), lambda i: (i, 0, 0)),
                pl.BlockSpec((_BP, K, 8), lambda i: (i, 0, 0))]
    args = [g1, g2]
    if has_e:
        in_specs += [pl.BlockSpec((_BP, K), lambda i: (i, 0)),
                     pl.BlockSpec((_BP, K), lambda i: (i, 0))]
        args += [e[0], e[1]]
    in_specs.append(pl.BlockSpec((H, 32), lambda i: (0, 0)))
    args.append(_headmask())
    in_specs.append(pl.BlockSpec(memory_space=pltpu.SMEM))
    args.append(p)
    return pl.pallas_call(
        body,
        grid=(N // _BP,),
        in_specs=in_specs,
        out_specs=[pl.BlockSpec((_BP, D), lambda i: (i, 0)),
                   pl.BlockSpec((1, D), lambda i: (0, 0)),
                   pl.BlockSpec((1, D), lambda i: (0, 0))],
        out_shape=[jax.ShapeDtypeStruct((N, D), jnp.float32),
                   jax.ShapeDtypeStruct((1, D), jnp.float32),
                   jax.ShapeDtypeStruct((1, D), jnp.float32)],
    )(*args)


def _final_body(y_ref, sc_ref, sh_ref, wr_ref, b_ref, o_ref):
    yn = y_ref[...] * sc_ref[...] + sh_ref[...]
    o_ref[...] = (jnp.sum(_r16(yn) * wr_ref[...], axis=1, keepdims=True)
                  + b_ref[...])


def _final_call(y, sc, sh, wr, b):
    vec = pl.BlockSpec((1, D), lambda i: (0, 0))
    return pl.pallas_call(
        _final_body,
        grid=(N // _BB,),
        in_specs=[pl.BlockSpec((_BB, D), lambda i: (i, 0)),
                  vec, vec, vec,
                  pl.BlockSpec((1, 1), lambda i: (0, 0))],
        out_specs=pl.BlockSpec((_BB, 1), lambda i: (i, 0)),
        out_shape=jax.ShapeDtypeStruct((N, 1), jnp.float32),
    )(y, sc, sh, wr, b)


def _att_mats(wa):
    rows = jnp.arange(D)
    heads = rows // F
    A = jnp.zeros((D, 14), jnp.float32)
    A = A.at[rows, heads].set(wa[rows % F, 0])
    A = A.at[rows, 3 + heads].set(wa[F + rows % F, 0])
    Ad = jnp.zeros((D, 8), jnp.float32)
    Ad = Ad.at[rows, heads].set(wa[F + rows % F, 0])
    return _r16(A), _r16(Ad)


def _headmask():
    R = jnp.zeros((H, 32), jnp.float32)
    rows = jnp.arange(D)
    R = R.at[rows // F, rows].set(1.0)
    return R


def _bn_fold(s, q, g, b):
    mu = s / N
    var = q / N - mu * mu
    scale = (g / jnp.sqrt(var[0] + EPS)).reshape(1, D)
    shift = (b - mu[0] * scale[0]).reshape(1, D)
    return scale, shift


def kernel(x, edge_index, e, W_node0, b_node0, W_att0, b_att0,
           W_node1, b_node1, W_att1, b_att1, W_node2, b_node2, W_att2,
           b_att2, bn0_g, bn0_b, bn1_g, bn1_b, bn2_g, bn2_b, bn3_g, bn3_b,
           W_fc, b_fc):
    src = edge_index[0]
    dst = edge_index[1]
    pad = EP - E
    sidx = jnp.pad(src, (0, pad)).reshape(NW, ROWS, CH)
    didx = jnp.pad(dst, (0, pad)).reshape(NW, ROWS, CH)
    e0 = e[:, 0].reshape(N, K)
    e1 = e[:, 1].reshape(N, K)

    s0, q0 = _stats_call(x)
    mu0 = s0[0, 0] / N
    var0 = q0[0, 0] / N - mu0 * mu0
    a0 = bn0_g[0] / jnp.sqrt(var0 + EPS)
    c0 = bn0_b[0] - mu0 * a0
    A0, Ad0 = _att_mats(W_att0)
    x0, T, Td = _pre0_call(x, jnp.stack([a0, c0]), W_node0,
                           b_node0.reshape(1, D), A0, Ad0)

    g1, g2 = _gather_call(T, Td, sidx, didx)
    p0 = jnp.stack([b_att0[0], _r16(W_att0[12, 0]), _r16(W_att0[13, 0])])
    y1, s1, q1 = _post_call(g1[:E].reshape(N, K, 32),
                            g2[:E].reshape(N, K, 8), (e0, e1), p0)
    sc1, sh1 = _bn_fold(s1, q1, bn1_g, bn1_b)

    A1, Ad1 = _att_mats(W_att1)
    T, Td = _pre_call(1, x0, [y1], [sc1, sh1], _r16(W_node1[0:1, :]),
                      [_r16(W_node1[1:1 + D, :])],
                      b_node1.reshape(1, D), A1, Ad1)
    g1, g2 = _gather_call(T, Td, sidx, didx)
    p1 = jnp.stack([b_att1[0], jnp.float32(0), jnp.float32(0)])
    y2, s2, q2 = _post_call(g1[:E].reshape(N, K, 32),
                            g2[:E].reshape(N, K, 8), None, p1)
    sc2, sh2 = _bn_fold(s2, q2, bn2_g, bn2_b)

    A2, Ad2 = _att_mats(W_att2)
    T, Td = _pre_call(2, x0, [y1, y2], [sc1, sh1, sc2, sh2],
                      _r16(W_node2[0:1, :]),
                      [_r16(W_node2[1:1 + D, :]),
                       _r16(W_node2[1 + D:1 + 2 * D, :])],
                      b_node2.reshape(1, D), A2, Ad2)
    g1, g2 = _gather_call(T, Td, sidx, didx)
    p2 = jnp.stack([b_att2[0], jnp.float32(0), jnp.float32(0)])
    y3, s3, q3 = _post_call(g1[:E].reshape(N, K, 32),
                            g2[:E].reshape(N, K, 8), None, p2)
    sc3, sh3 = _bn_fold(s3, q3, bn3_g, bn3_b)

    return _final_call(y3, sc3, sh3, _r16(W_fc[:, 0].reshape(1, D)),
                       b_fc.reshape(1, 1))


# speedup vs baseline: 3.4084x; 3.4084x over previous
import functools

import jax
import jax.numpy as jnp
from jax import lax
from jax.experimental import pallas as pl
from jax.experimental.pallas import tpu as pltpu
from jax.experimental.pallas import tpu_sc as plsc

N = 50000
K = 16
H = 3
F = 6
E = N * K
D = H * F

NC = 2
NS = 16
NW = NC * NS
CH = 128
ROWS = -(-E // (NW * CH))
EP = NW * ROWS * CH

EPS = 1e-5


def _r16(v):
    return v.astype(jnp.bfloat16).astype(jnp.float32)


def _gather_body(t32, td, sidx, didx, g1, g2, idx1_v, idx2_v,
                 r1a, r2a, r1b, r2b, r1c, r2c, r1d, r2d,
                 sem_a, sem_b, sem_c, sem_d):
    c = lax.axis_index("c")
    s = lax.axis_index("s")
    wid = s * NC + c
    pltpu.sync_copy(sidx.at[wid], idx1_v)
    pltpu.sync_copy(didx.at[wid], idx2_v)

    def fire(j, r1, r2, sem):
        pltpu.async_copy(t32.at[idx1_v.at[j]], r1, sem)
        pltpu.async_copy(td.at[idx2_v.at[j]], r2, sem)

    def drain_wb(j, r1, r2, sem):
        pltpu.make_async_copy(t32.at[idx1_v.at[j]], r1, sem).wait()
        pltpu.make_async_copy(td.at[idx2_v.at[j]], r2, sem).wait()
        base = (wid * ROWS + j) * CH
        pltpu.sync_copy(r1, g1.at[pl.ds(base, CH)])
        pltpu.sync_copy(r2, g2.at[pl.ds(base, CH)])

    bufs = [(r1a, r2a, sem_a), (r1b, r2b, sem_b),
            (r1c, r2c, sem_c), (r1d, r2d, sem_d)]
    for b in range(3):
        fire(b, *bufs[b])

    @pl.loop(0, ROWS, step=4)
    def _(j):
        for b in range(4):
            jj = j + b

            @pl.when(jj + 3 < ROWS)
            def _():
                fire(jj + 3, *bufs[(b + 3) % 4])

            drain_wb(jj, *bufs[b])


@functools.cache
def _make_gather():
    return pl.kernel(
        _gather_body,
        out_type=(jax.ShapeDtypeStruct((EP, 32), jnp.float32),
                  jax.ShapeDtypeStruct((EP, 8), jnp.float32)),
        mesh=plsc.VectorSubcoreMesh(core_axis_name="c", subcore_axis_name="s"),
        scratch_types=[
            pltpu.VMEM((ROWS, CH), jnp.int32),
            pltpu.VMEM((ROWS, CH), jnp.int32),
            pltpu.VMEM((CH, 32), jnp.float32),
            pltpu.VMEM((CH, 8), jnp.float32),
            pltpu.VMEM((CH, 32), jnp.float32),
            pltpu.VMEM((CH, 8), jnp.float32),
            pltpu.VMEM((CH, 32), jnp.float32),
            pltpu.VMEM((CH, 8), jnp.float32),
            pltpu.VMEM((CH, 32), jnp.float32),
            pltpu.VMEM((CH, 8), jnp.float32),
            pltpu.SemaphoreType.DMA,
            pltpu.SemaphoreType.DMA,
            pltpu.SemaphoreType.DMA,
            pltpu.SemaphoreType.DMA,
        ],
        compiler_params=pltpu.CompilerParams(use_tc_tiling_on_sc=False),
    )


def _gather_call(t32, td, sidx, didx):
    return _make_gather()(t32, td, sidx, didx)


_BB = 2000
_BP = 400


def _stats_body(x_ref, s_ref, q_ref):
    @pl.when(pl.program_id(0) == 0)
    def _():
        s_ref[...] = jnp.zeros_like(s_ref)
        q_ref[...] = jnp.zeros_like(q_ref)

    xb = x_ref[...]
    s_ref[...] += jnp.sum(xb, axis=0, keepdims=True)
    q_ref[...] += jnp.sum(xb * xb, axis=0, keepdims=True)


def _stats_call(x):
    return pl.pallas_call(
        _stats_body,
        grid=(N // _BB,),
        in_specs=[pl.BlockSpec((_BB, 1), lambda i: (i, 0))],
        out_specs=[pl.BlockSpec((1, 1), lambda i: (0, 0)),
                   pl.BlockSpec((1, 1), lambda i: (0, 0))],
        out_shape=[jax.ShapeDtypeStruct((1, 1), jnp.float32),
                   jax.ShapeDtypeStruct((1, 1), jnp.float32)],
    )(x)


def _dot_hi(a, b):
    return jnp.dot(a, b, preferred_element_type=jnp.float32,
                   precision=lax.Precision.HIGHEST)


def _dot_bf(a, b):
    return jnp.dot(a, b, preferred_element_type=jnp.float32)


def _pre0_body(x_ref, a_ref, wx_ref, b_ref, Ar_ref, Adr_ref,
               x0_ref, t_ref, td_ref):
    x0 = x_ref[...] * a_ref[0] + a_ref[1]
    x0_ref[...] = x0
    xe = x0 * wx_ref[...] + b_ref[...]
    xeh = _r16(xe)
    t_ref[...] = jnp.concatenate([xe, _dot_bf(xeh, Ar_ref[...])], axis=1)
    td_ref[...] = _dot_bf(xeh, Adr_ref[...])


def _pre0_call(x, a, wx, b, Ar, Adr):
    return pl.pallas_call(
        _pre0_body,
        grid=(N // _BB,),
        in_specs=[
            pl.BlockSpec((_BB, 1), lambda i: (i, 0)),
            pl.BlockSpec(memory_space=pltpu.SMEM),
            pl.BlockSpec((1, D), lambda i: (0, 0)),
            pl.BlockSpec((1, D), lambda i: (0, 0)),
            pl.BlockSpec((D, 14), lambda i: (0, 0)),
            pl.BlockSpec((D, 8), lambda i: (0, 0)),
        ],
        out_specs=[pl.BlockSpec((_BB, 1), lambda i: (i, 0)),
                   pl.BlockSpec((_BB, 32), lambda i: (i, 0)),
                   pl.BlockSpec((_BB, 8), lambda i: (i, 0))],
        out_shape=[jax.ShapeDtypeStruct((N, 1), jnp.float32),
                   jax.ShapeDtypeStruct((N, 32), jnp.float32),
                   jax.ShapeDtypeStruct((N, 8), jnp.float32)],
    )(x, a, wx, b, Ar, Adr)


def _pre_body(nf, refs):
    i = 0
    x0_ref = refs[i]; i += 1
    y_refs = refs[i:i + nf]; i += nf
    ss_refs = refs[i:i + 2 * nf]; i += 2 * nf
    wxr_ref = refs[i]; i += 1
    w_refs = refs[i:i + nf]; i += nf
    b_ref = refs[i]; i += 1
    Ar_ref = refs[i]; Adr_ref = refs[i + 1]; i += 2
    t_ref = refs[i]; td_ref = refs[i + 1]
    xe = _r16(x0_ref[...]) * wxr_ref[...] + b_ref[...]
    for g in range(nf):
        yn = y_refs[g][...] * ss_refs[2 * g][...] + ss_refs[2 * g + 1][...]
        xe += _dot_bf(_r16(yn), w_refs[g][...])
    xeh = _r16(xe)
    t_ref[...] = jnp.concatenate([xe, _dot_bf(xeh, Ar_ref[...])], axis=1)
    td_ref[...] = _dot_bf(xeh, Adr_ref[...])


def _pre_call(nf, x0, ys, sss, wxr, ws, b, Ar, Adr):
    body = lambda *refs: _pre_body(nf, refs)
    vec = pl.BlockSpec((1, D), lambda i: (0, 0))
    return pl.pallas_call(
        body,
        grid=(N // _BB,),
        in_specs=(
            [pl.BlockSpec((_BB, 1), lambda i: (i, 0))]
            + [pl.BlockSpec((_BB, D), lambda i: (i, 0))] * nf
            + [vec] * (2 * nf)
            + [vec]
            + [pl.BlockSpec((D, D), lambda i: (0, 0))] * nf
            + [vec,
               pl.BlockSpec((D, 14), lambda i: (0, 0)),
               pl.BlockSpec((D, 8), lambda i: (0, 0))]
        ),
        out_specs=[pl.BlockSpec((_BB, 32), lambda i: (i, 0)),
                   pl.BlockSpec((_BB, 8), lambda i: (i, 0))],
        out_shape=[jax.ShapeDtypeStruct((N, 32), jnp.float32),
                   jax.ShapeDtypeStruct((N, 8), jnp.float32)],
    )(x0, *ys, *sss, wxr, *ws, b, Ar, Adr)


def _post_body(has_e, refs):
    if has_e:
        g1_ref, g2_ref, e0_ref, e1_ref, R_ref, p_ref, y_ref, s_ref, q_ref = refs
    else:
        g1_ref, g2_ref, R_ref, p_ref, y_ref, s_ref, q_ref = refs
    g1 = g1_ref[...]
    att = g1[:, :, D:D + H] + g2_ref[...][:, :, 0:3] + p_ref[0]
    if has_e:
        ae = _r16(e0_ref[...]) * p_ref[1] + _r16(e1_ref[...]) * p_ref[2]
        att += ae[:, :, None]
    m = jnp.max(att, axis=1, keepdims=True)
    w = jnp.exp(att - m)
    wn = w / jnp.sum(w, axis=1, keepdims=True)
    wfull = jax.lax.dot_general(
        wn, R_ref[...], (((2,), (0,)), ((), ())),
        preferred_element_type=jnp.float32,
        precision=lax.Precision.HIGHEST)
    y = jnp.sum(wfull * g1, axis=1)[:, 0:D] * (1.0 / K)
    y_ref[...] = y

    @pl.when(pl.program_id(0) == 0)
    def _():
        s_ref[...] = jnp.zeros_like(s_ref)
        q_ref[...] = jnp.zeros_like(q_ref)

    s_ref[...] += jnp.sum(y, axis=0, keepdims=True)
    q_ref[...] += jnp.sum(y * y, axis=0, keepdims=True)


def _post_call(g1, g2, e, p):
    has_e = e is not None
    body = lambda *refs: _post_body(has_e, refs)
    in_specs = [pl.BlockSpec((_BP, K, 32), lambda i: (i, 0, 0)),
                pl.BlockSpec((_BP, K, 8), lambda i: (i, 0, 0))]
    args = [g1, g2]
    if has_e:
        in_specs += [pl.BlockSpec((_BP, K), lambda i: (i, 0)),
                     pl.BlockSpec((_BP, K), lambda i: (i, 0))]
        args += [e[0], e[1]]
    in_specs.append(pl.BlockSpec((H, 32), lambda i: (0, 0)))
    args.append(_headmask())
    in_specs.append(pl.BlockSpec(memory_space=pltpu.SMEM))
    args.append(p)
    return pl.pallas_call(
        body,
        grid=(N // _BP,),
        in_specs=in_specs,
        out_specs=[pl.BlockSpec((_BP, D), lambda i: (i, 0)),
                   pl.BlockSpec((1, D), lambda i: (0, 0)),
                   pl.BlockSpec((1, D), lambda i: (0, 0))],
        out_shape=[jax.ShapeDtypeStruct((N, D), jnp.float32),
                   jax.ShapeDtypeStruct((1, D), jnp.float32),
                   jax.ShapeDtypeStruct((1, D), jnp.float32)],
    )(*args)


def _final_body(y_ref, sc_ref, sh_ref, wr_ref, b_ref, o_ref):
    yn = y_ref[...] * sc_ref[...] + sh_ref[...]
    o_ref[...] = (jnp.sum(_r16(yn) * wr_ref[...], axis=1, keepdims=True)
                  + b_ref[...])


def _final_call(y, sc, sh, wr, b):
    vec = pl.BlockSpec((1, D), lambda i: (0, 0))
    return pl.pallas_call(
        _final_body,
        grid=(N // _BB,),
        in_specs=[pl.BlockSpec((_BB, D), lambda i: (i, 0)),
                  vec, vec, vec,
                  pl.BlockSpec((1, 1), lambda i: (0, 0))],
        out_specs=pl.BlockSpec((_BB, 1), lambda i: (i, 0)),
        out_shape=jax.ShapeDtypeStruct((N, 1), jnp.float32),
    )(y, sc, sh, wr, b)


def _att_mats(wa):
    rows = jnp.arange(D)
    heads = rows // F
    A = jnp.zeros((D, 14), jnp.float32)
    A = A.at[rows, heads].set(wa[rows % F, 0])
    A = A.at[rows, 3 + heads].set(wa[F + rows % F, 0])
    Ad = jnp.zeros((D, 8), jnp.float32)
    Ad = Ad.at[rows, heads].set(wa[F + rows % F, 0])
    return _r16(A), _r16(Ad)


def _headmask():
    R = jnp.zeros((H, 32), jnp.float32)
    rows = jnp.arange(D)
    R = R.at[rows // F, rows].set(1.0)
    return R


def _bn_fold(s, q, g, b):
    mu = s / N
    var = q / N - mu * mu
    scale = (g / jnp.sqrt(var[0] + EPS)).reshape(1, D)
    shift = (b - mu[0] * scale[0]).reshape(1, D)
    return scale, shift


def kernel(x, edge_index, e, W_node0, b_node0, W_att0, b_att0,
           W_node1, b_node1, W_att1, b_att1, W_node2, b_node2, W_att2,
           b_att2, bn0_g, bn0_b, bn1_g, bn1_b, bn2_g, bn2_b, bn3_g, bn3_b,
           W_fc, b_fc):
    src = edge_index[0]
    dst = edge_index[1]
    pad = EP - E
    sidx = jnp.pad(src, (0, pad)).reshape(NW, ROWS, CH)
    didx = jnp.pad(dst, (0, pad)).reshape(NW, ROWS, CH)
    e0 = e[:, 0].reshape(N, K)
    e1 = e[:, 1].reshape(N, K)

    s0, q0 = _stats_call(x)
    mu0 = s0[0, 0] / N
    var0 = q0[0, 0] / N - mu0 * mu0
    a0 = bn0_g[0] / jnp.sqrt(var0 + EPS)
    c0 = bn0_b[0] - mu0 * a0
    A0, Ad0 = _att_mats(W_att0)
    x0, T, Td = _pre0_call(x, jnp.stack([a0, c0]), W_node0,
                           b_node0.reshape(1, D), A0, Ad0)

    g1, g2 = _gather_call(T, Td, sidx, didx)
    p0 = jnp.stack([b_att0[0], _r16(W_att0[12, 0]), _r16(W_att0[13, 0])])
    y1, s1, q1 = _post_call(g1[:E].reshape(N, K, 32),
                            g2[:E].reshape(N, K, 8), (e0, e1), p0)
    sc1, sh1 = _bn_fold(s1, q1, bn1_g, bn1_b)

    A1, Ad1 = _att_mats(W_att1)
    T, Td = _pre_call(1, x0, [y1], [sc1, sh1], _r16(W_node1[0:1, :]),
                      [_r16(W_node1[1:1 + D, :])],
                      b_node1.reshape(1, D), A1, Ad1)
    g1, g2 = _gather_call(T, Td, sidx, didx)
    p1 = jnp.stack([b_att1[0], jnp.float32(0), jnp.float32(0)])
    y2, s2, q2 = _post_call(g1[:E].reshape(N, K, 32),
                            g2[:E].reshape(N, K, 8), None, p1)
    sc2, sh2 = _bn_fold(s2, q2, bn2_g, bn2_b)

    A2, Ad2 = _att_mats(W_att2)
    T, Td = _pre_call(2, x0, [y1, y2], [sc1, sh1, sc2, sh2],
                      _r16(W_node2[0:1, :]),
                      [_r16(W_node2[1:1 + D, :]),
                       _r16(W_node2[1 + D:1 + 2 * D, :])],
                      b_node2.reshape(1, D), A2, Ad2)
    g1, g2 = _gather_call(T, Td, sidx, didx)
    p2 = jnp.stack([b_att2[0], jnp.float32(0), jnp.float32(0)])
    y3, s3, q3 = _post_call(g1[:E].reshape(N, K, 32),
                            g2[:E].reshape(N, K, 8), None, p2)
    sc3, sh3 = _bn_fold(s3, q3, bn3_g, bn3_b)

    return _final_call(y3, sc3, sh3, _r16(W_fc[:, 0].reshape(1, D)),
                       b_fc.reshape(1, 1))
